# baseline (speedup 1.0000x reference)
import functools

import jax
import jax.numpy as jnp
from jax import lax
from jax.experimental import pallas as pl
from jax.experimental.pallas import tpu as pltpu
from jax.experimental.pallas import tpu_sc as plsc

EMBED_DIM = 600
MAIN_DIM = 512
TAIL_DIM = 88
TAIL_PAD = 128
SEQ = 200
BATCH = 1024
BBLK = 128
NBB = BATCH // BBLK
NUM_WORKERS = 32
W_CHUNKS = SEQ * NBB // NUM_WORKERS


def _embed_gather(idx3, table, tail):
    mesh = plsc.VectorSubcoreMesh(core_axis_name="c", subcore_axis_name="s")

    @functools.partial(
        pl.kernel,
        mesh=mesh,
        compiler_params=pltpu.CompilerParams(needs_layout_passes=False),
        out_type=jax.ShapeDtypeStruct((SEQ, EMBED_DIM, BATCH), jnp.float32),
        scratch_types=[
            pltpu.VMEM((BBLK,), jnp.int32),
            pltpu.VMEM((BBLK, TAIL_PAD), jnp.float32),
            pltpu.VMEM((BBLK, TAIL_PAD), jnp.float32),
            pltpu.VMEM((TAIL_PAD, BBLK), jnp.float32),
            pltpu.VMEM((TAIL_PAD, BBLK), jnp.float32),
            pltpu.SemaphoreType.DMA((2,)),
            pltpu.SemaphoreType.DMA((2,)),
            pltpu.SemaphoreType.DMA,
        ],
    )
    def k(idx_hbm, table_hbm, tail_hbm, out_hbm,
          idx_v, in0, in1, st0, st1, semg, semo, semt):
        wid = lax.axis_index("s") * 2 + lax.axis_index("c")
        ins = (in0, in1)
        sts = (st0, st1)
        lane16 = lax.iota(jnp.int32, 16)
        rvecs = [lane16 + 16 * kk for kk in range(8)]

        def gather_src(p):
            if p < 4:
                return table_hbm.at[idx_v, pl.ds(p * TAIL_PAD, TAIL_PAD)]
            return tail_hbm.at[idx_v]

        def out_dst(p, l, bb):
            nrows = TAIL_PAD if p < 4 else TAIL_DIM
            return out_hbm.at[l, pl.ds(p * TAIL_PAD, nrows), pl.ds(bb * BBLK, BBLK)]

        def body(c, carry):
            chunk = wid * W_CHUNKS + c
            l = lax.div(chunk, NBB)
            bb = lax.rem(chunk, NBB)
            pltpu.sync_copy(idx_hbm.at[l, bb], idx_v)
            pltpu.async_copy(gather_src(0), ins[0], semg.at[0])

            for p in range(5):
                b = p % 2
                if p < 4:
                    pltpu.async_copy(gather_src(p + 1), ins[1 - b], semg.at[1 - b])
                pltpu.make_async_copy(gather_src(p), ins[b], semg.at[b]).wait()

                if p >= 2:
                    pltpu.make_async_copy(
                        sts[b], out_dst(p - 2, l, bb), semo.at[b]
                    ).wait()
                elif p == 0:

                    @pl.when(c > 0)
                    def _():
                        pltpu.make_async_copy(
                            st0.at[pl.ds(0, TAIL_DIM)], out_dst(4, l, bb), semt
                        ).wait()

                else:

                    @pl.when(c > 0)
                    def _():
                        pltpu.make_async_copy(
                            sts[1], out_dst(3, l, bb), semo.at[1]
                        ).wait()

                nrows = TAIL_PAD if p < 4 else TAIL_DIM
                src = ins[b]
                dst = sts[b]

                def transpose_row(dd, cc, src=src, dst=dst):
                    dvec = jnp.full((16,), dd, jnp.int32)
                    for kk in range(8):
                        dst[dd, pl.ds(16 * kk, 16)] = plsc.load_gather(
                            src, [rvecs[kk], dvec]
                        )
                    return cc

                lax.fori_loop(0, nrows, transpose_row, 0)

                if p < 4:
                    pltpu.async_copy(sts[b], out_dst(p, l, bb), semo.at[b])
                else:
                    pltpu.async_copy(
                        st0.at[pl.ds(0, TAIL_DIM)], out_dst(4, l, bb), semt
                    )
            return carry

        lax.fori_loop(0, W_CHUNKS, body, 0)

        last = wid * W_CHUNKS + W_CHUNKS - 1
        l_last = lax.div(last, NBB)
        bb_last = lax.rem(last, NBB)
        pltpu.make_async_copy(sts[1], out_dst(3, l_last, bb_last), semo.at[1]).wait()
        pltpu.make_async_copy(
            st0.at[pl.ds(0, TAIL_DIM)], out_dst(4, l_last, bb_last), semt
        ).wait()

    return k(idx3, table, tail)


def _tail_copy_body(tab_ref, tail_ref):
    tail_ref[...] = tab_ref[...]


def _build_tail(table):
    vocab = table.shape[0]
    rows = 1000
    return pl.pallas_call(
        _tail_copy_body,
        grid=(vocab // rows,),
        in_specs=[
            pl.BlockSpec((rows, TAIL_PAD), lambda i: (i, MAIN_DIM // TAIL_PAD))
        ],
        out_specs=pl.BlockSpec((rows, TAIL_PAD), lambda i: (i, 0)),
        out_shape=jax.ShapeDtypeStruct((vocab, TAIL_PAD), jnp.float32),
    )(table)


def kernel(batch, table):
    idx3 = batch.T.reshape(SEQ, NBB, BBLK)
    tail = _build_tail(table)
    out = _embed_gather(idx3, table, tail)
    return jnp.transpose(out, (2, 0, 1))


# speedup vs baseline: 3.2285x; 3.2285x over previous
import functools

import jax
import jax.numpy as jnp
from jax import lax
from jax.experimental import pallas as pl
from jax.experimental.pallas import tpu as pltpu
from jax.experimental.pallas import tpu_sc as plsc

EMBED_DIM = 600
MAIN_DIM = 512
TAIL_DIM = 88
TAIL_PAD = 128
NUM_WORKERS = 32
ROWS = 64
CHUNKS = 100


def _embed_gather(idx3d, table, tail):
    mesh = plsc.VectorSubcoreMesh(core_axis_name="c", subcore_axis_name="s")

    @functools.partial(
        pl.kernel,
        mesh=mesh,
        compiler_params=pltpu.CompilerParams(needs_layout_passes=False),
        out_type=jax.ShapeDtypeStruct(
            (NUM_WORKERS, CHUNKS, ROWS, EMBED_DIM), jnp.float32
        ),
        scratch_types=[
            pltpu.VMEM((CHUNKS, ROWS), jnp.int32),
            pltpu.VMEM((2, ROWS, EMBED_DIM), jnp.float32),
            pltpu.VMEM((2, ROWS, TAIL_PAD), jnp.float32),
            pltpu.SemaphoreType.DMA((2,)),
            pltpu.SemaphoreType.DMA((2,)),
        ],
    )
    def k(idx_hbm, table_hbm, tail_hbm, out_hbm, idx_v, stage_v, tail_v, sems, sems_t):
        wid = lax.axis_index("s") * 2 + lax.axis_index("c")
        pltpu.sync_copy(idx_hbm.at[wid], idx_v)

        def start_gathers(g, b):
            pltpu.async_copy(
                table_hbm.at[idx_v.at[g], pl.ds(0, MAIN_DIM)],
                stage_v.at[b, :, pl.ds(0, MAIN_DIM)],
                sems.at[b],
            )
            pltpu.async_copy(tail_hbm.at[idx_v.at[g]], tail_v.at[b], sems_t.at[b])

        start_gathers(0, 0)

        def body(g, carry):
            b = lax.rem(g, 2)
            nb = lax.rem(g + 1, 2)

            @pl.when(g + 1 < CHUNKS)
            def _():
                start_gathers(g + 1, nb)

            pltpu.make_async_copy(
                table_hbm.at[idx_v.at[g], pl.ds(0, MAIN_DIM)],
                stage_v.at[b, :, pl.ds(0, MAIN_DIM)],
                sems.at[b],
            ).wait()
            pltpu.make_async_copy(
                tail_hbm.at[idx_v.at[g]], tail_v.at[b], sems_t.at[b]
            ).wait()

            lane = lax.iota(jnp.int32, 16)
            tail_mask = lane < (TAIL_DIM - 80)
            col_idx = jnp.minimum(MAIN_DIM + 80 + lane, EMBED_DIM - 1)

            def merge_row(r, c):
                for i in range(5):
                    stage_v[b, r, pl.ds(MAIN_DIM + i * 16, 16)] = tail_v[
                        b, r, pl.ds(i * 16, 16)
                    ]
                plsc.store_scatter(
                    stage_v,
                    [jnp.full((16,), b, jnp.int32), jnp.full((16,), r, jnp.int32),
                     col_idx],
                    tail_v[b, r, pl.ds(80, 16)],
                    mask=tail_mask,
                )
                return c

            lax.fori_loop(0, ROWS, merge_row, 0)

            pltpu.sync_copy(stage_v.at[b], out_hbm.at[wid, g])
            return carry

        lax.fori_loop(0, CHUNKS, body, 0)

    return k(idx3d, table, tail)


def kernel(batch, table):
    B, L = batch.shape
    idx3d = batch.reshape(NUM_WORKERS, CHUNKS, ROWS)
    tail = jnp.pad(table[:, MAIN_DIM:], ((0, 0), (0, TAIL_PAD - TAIL_DIM)))
    out = _embed_gather(idx3d, table, tail)
    return out.reshape(B, L, EMBED_DIM)
